# baseline (speedup 1.0000x reference)
import functools

import jax
import jax.numpy as jnp
from jax import lax
from jax.experimental import pallas as pl
from jax.experimental.pallas import tpu as pltpu
from jax.experimental.pallas import tpu_sc as plsc

B = 16384
D_H = 128
N_BIG = 2


_DN_T = (((0,), (0,)), ((), ()))
_DN_T1 = (((1,), (1,)), ((), ()))
_DN_01 = (((0,), (1,)), ((), ()))

_KBLK = 256
_KGRID = 7


def _fold_big_body(tblt_ref, w_ref, extra_ref, out_ref):
    k = pl.program_id(0)
    part = lax.dot_general(tblt_ref[...], w_ref[...], _DN_T,
                           preferred_element_type=jnp.float32)

    @pl.when(k == 0)
    def _():
        out_ref[...] = extra_ref[...] + part

    @pl.when(k > 0)
    def _():
        out_ref[...] = out_ref[...] + part


def _fold_big(tbl_t, w, extra):
    rows = tbl_t.shape[1]
    return pl.pallas_call(
        _fold_big_body,
        grid=(_KGRID,),
        in_specs=[
            pl.BlockSpec((_KBLK, rows), lambda i: (i, 0)),
            pl.BlockSpec((_KBLK, D_H), lambda i: (i, 0)),
            pl.BlockSpec((rows, D_H), lambda i: (0, 0)),
        ],
        out_specs=pl.BlockSpec((rows, D_H), lambda i: (0, 0)),
        out_shape=jax.ShapeDtypeStruct((rows, D_H), jnp.float32),
    )(tbl_t, w, extra)


def _fold_small_body(m_ref, s_ref, ct_ref, at_ref, w1t_ref,
                     waux_ref, p3_ref, extra_ref):
    w1t = w1t_ref[...]
    p0t = lax.dot_general(w1t[:, 0:5], m_ref[...], _DN_T1,
                          preferred_element_type=jnp.float32)
    p1t = lax.dot_general(w1t[:, 5:9], s_ref[...], _DN_T1,
                          preferred_element_type=jnp.float32)
    waux_ref[...] = jnp.concatenate([p0t, p1t, w1t[:, 2090:2093]], axis=1)
    p3_ref[...] = lax.dot_general(ct_ref[...], w1t[:, 1808:2090], _DN_01,
                                  preferred_element_type=jnp.float32)
    extra_ref[...] = lax.dot_general(at_ref[...], w1t[:, 1801:1808], _DN_01,
                                     preferred_element_type=jnp.float32)


def _fold_small(m, s, c_t, a_tail_t, w1t):
    return pl.pallas_call(
        _fold_small_body,
        out_shape=(
            jax.ShapeDtypeStruct((D_H, 12), jnp.float32),
            jax.ShapeDtypeStruct((c_t.shape[1], D_H), jnp.float32),
            jax.ShapeDtypeStruct((a_tail_t.shape[1], D_H), jnp.float32),
        ),
    )(m, s, c_t, a_tail_t, w1t)


_NW = 32
_BPW = B // _NW
_CHUNK = 128
_NCH = _BPW // _CHUNK


_SETS = 3


@functools.cache
def _make_sc_gather_sum():
    @functools.partial(
        pl.kernel,
        mesh=plsc.VectorSubcoreMesh(core_axis_name="c", subcore_axis_name="s"),
        out_type=jax.ShapeDtypeStruct((B, D_H), jnp.float32),
        scratch_types=[
            pltpu.VMEM((N_BIG, _BPW), jnp.int32),
            pltpu.VMEM((_SETS, N_BIG, _CHUNK, D_H), jnp.float32),
            pltpu.SemaphoreType.DMA,
            pltpu.SemaphoreType.DMA,
            pltpu.SemaphoreType.DMA,
            pltpu.SemaphoreType.DMA,
        ],
    )
    def _sc_gather_sum(t2_hbm, t3_hbm, xctf_hbm, out_hbm,
                       idx_v, rows_v, sg0, sg1, sg2, so):
        wid = lax.axis_index("s") * 2 + lax.axis_index("c")
        pltpu.sync_copy(xctf_hbm.at[pl.ds(2 * B + wid * _BPW, _BPW)],
                        idx_v.at[0])
        pltpu.sync_copy(xctf_hbm.at[pl.ds(3 * B + wid * _BPW, _BPW)],
                        idx_v.at[1])
        tbls = (t2_hbm, t3_hbm)
        gsems = (sg0, sg1, sg2)
        handles = [None] * _SETS
        out_h = [None] * _SETS

        def issue(ch):
            s = ch % _SETS
            if out_h[s] is not None:
                out_h[s].wait()
                out_h[s] = None
            handles[s] = [
                pltpu.async_copy(
                    tbls[t].at[idx_v.at[t, pl.ds(ch * _CHUNK, _CHUNK)]],
                    rows_v.at[s, t], gsems[s])
                for t in range(N_BIG)
            ]

        for ch in range(min(_SETS, _NCH)):
            issue(ch)
        for ch in range(_NCH):
            s = ch % _SETS
            for h in handles[s]:
                h.wait()

            def _acc_row(r, carry):
                for j in range(D_H // 16):
                    sl = pl.ds(j * 16, 16)
                    rows_v[s, 0, r, sl] = rows_v[s, 0, r, sl] + rows_v[s, 1, r, sl]
                return carry

            lax.fori_loop(0, _CHUNK, _acc_row, 0)
            out_h[s] = pltpu.async_copy(
                rows_v.at[s, 0],
                out_hbm.at[pl.ds(wid * _BPW + ch * _CHUNK, _CHUNK)], so)
            if ch + _SETS < _NCH:
                issue(ch + _SETS)
        for s in range(_SETS):
            if out_h[s] is not None:
                out_h[s].wait()

    return _sc_gather_sum


_MLP_BLK = 2048


_DN_OUT = (((0,), (1,)), ((), ()))


def _mlp_body(g_ref, xct_ref, xnt_ref, waux_ref, b1_ref, w2_ref,
              b2_ref, wo_ref, bo_ref, out_ref):
    xct = xct_ref[...]
    oh0 = (xct[0:1, :] == lax.broadcasted_iota(jnp.int32, (5, _MLP_BLK), 0))
    oh1 = (xct[1:2, :] == lax.broadcasted_iota(jnp.int32, (4, _MLP_BLK), 0))
    aux_t = jnp.concatenate([oh0.astype(jnp.float32), oh1.astype(jnp.float32),
                             xnt_ref[...]], axis=0)
    h = g_ref[...] + lax.dot_general(
        aux_t, waux_ref[...], _DN_01,
        preferred_element_type=jnp.float32) + b1_ref[...]
    h = jnp.maximum(h, 0.0)
    h2 = jnp.dot(h, w2_ref[...], preferred_element_type=jnp.float32) + b2_ref[...]
    h2 = jnp.maximum(h2, 0.0)
    out_ref[...] = lax.dot_general(wo_ref[...], h2, _DN_OUT,
                                   preferred_element_type=jnp.float32) + bo_ref[...]


def _mlp(g, xc_t, xn_t, waux, b1, w2, b2, wo, bo):
    grid = (B // _MLP_BLK,)
    return pl.pallas_call(
        _mlp_body,
        grid=grid,
        in_specs=[
            pl.BlockSpec((_MLP_BLK, D_H), lambda i: (i, 0)),
            pl.BlockSpec((4, _MLP_BLK), lambda i: (0, i)),
            pl.BlockSpec((3, _MLP_BLK), lambda i: (0, i)),
            pl.BlockSpec((D_H, 12), lambda i: (0, 0)),
            pl.BlockSpec((1, D_H), lambda i: (0, 0)),
            pl.BlockSpec((D_H, 64), lambda i: (0, 0)),
            pl.BlockSpec((1, 64), lambda i: (0, 0)),
            pl.BlockSpec((64, 1), lambda i: (0, 0)),
            pl.BlockSpec((1, 1), lambda i: (0, 0)),
        ],
        out_specs=pl.BlockSpec((1, _MLP_BLK), lambda i: (0, i)),
        out_shape=jax.ShapeDtypeStruct((1, B), jnp.float32),
    )(g, xc_t, xn_t, waux, b1, w2, b2, wo, bo)


def kernel(x_cat, x_num, market_emb, ship_emb, order_city_emb,
           customer_city_emb, W1, b1, W2, b2, Wo, bo):
    a_t = order_city_emb.T
    w1t = W1.T
    w1c = W1[9:1808]

    w_aux, p3, extra = _fold_small(market_emb, ship_emb,
                                   customer_city_emb.T, a_t[1792:1799], w1t)
    p2 = _fold_big(a_t, w1c, extra)

    xct = x_cat.astype(jnp.int32).T

    g = _make_sc_gather_sum()(p2, p3, xct.reshape(-1))

    out_t = _mlp(g, xct, x_num.T, w_aux, b1.reshape(1, D_H), W2,
                 b2.reshape(1, 64), Wo, bo.reshape(1, 1))
    return out_t.T


# speedup vs baseline: 1.0399x; 1.0399x over previous
import functools

import jax
import jax.numpy as jnp
from jax import lax
from jax.experimental import pallas as pl
from jax.experimental.pallas import tpu as pltpu
from jax.experimental.pallas import tpu_sc as plsc

B = 16384
D_H = 128
N_BIG = 2


_DN_T = (((0,), (0,)), ((), ()))
_DN_T1 = (((1,), (1,)), ((), ()))
_DN_01 = (((0,), (1,)), ((), ()))

_KBLK = 256
_KGRID = 7


def _fold_big_body(tblt_ref, w_ref, extra_ref, out_ref):
    k = pl.program_id(0)
    part = lax.dot_general(tblt_ref[...], w_ref[...], _DN_T,
                           preferred_element_type=jnp.float32)

    @pl.when(k == 0)
    def _():
        out_ref[...] = extra_ref[...] + part

    @pl.when(k > 0)
    def _():
        out_ref[...] = out_ref[...] + part


def _fold_big(tbl_t, w, extra):
    rows = tbl_t.shape[1]
    return pl.pallas_call(
        _fold_big_body,
        grid=(_KGRID,),
        in_specs=[
            pl.BlockSpec((_KBLK, rows), lambda i: (i, 0)),
            pl.BlockSpec((_KBLK, D_H), lambda i: (i, 0)),
            pl.BlockSpec((rows, D_H), lambda i: (0, 0)),
        ],
        out_specs=pl.BlockSpec((rows, D_H), lambda i: (0, 0)),
        out_shape=jax.ShapeDtypeStruct((rows, D_H), jnp.float32),
    )(tbl_t, w, extra)


def _fold_small_body(m_ref, s_ref, ct_ref, at_ref, w1t_ref,
                     waux_ref, p3_ref, extra_ref):
    w1t = w1t_ref[...]
    p0t = lax.dot_general(w1t[:, 0:5], m_ref[...], _DN_T1,
                          preferred_element_type=jnp.float32)
    p1t = lax.dot_general(w1t[:, 5:9], s_ref[...], _DN_T1,
                          preferred_element_type=jnp.float32)
    waux_ref[...] = jnp.concatenate([p0t, p1t, w1t[:, 2090:2093]], axis=1)
    p3_ref[...] = lax.dot_general(ct_ref[...], w1t[:, 1808:2090], _DN_01,
                                  preferred_element_type=jnp.float32)
    extra_ref[...] = lax.dot_general(at_ref[...], w1t[:, 1801:1808], _DN_01,
                                     preferred_element_type=jnp.float32)


def _fold_small(m, s, c_t, a_tail_t, w1t):
    return pl.pallas_call(
        _fold_small_body,
        out_shape=(
            jax.ShapeDtypeStruct((D_H, 12), jnp.float32),
            jax.ShapeDtypeStruct((c_t.shape[1], D_H), jnp.float32),
            jax.ShapeDtypeStruct((a_tail_t.shape[1], D_H), jnp.float32),
        ),
    )(m, s, c_t, a_tail_t, w1t)


_NW = 32
_BPW = B // _NW
_CHUNK = 128
_NCH = _BPW // _CHUNK


_SETS = 3


@functools.cache
def _make_sc_gather_sum():
    @functools.partial(
        pl.kernel,
        mesh=plsc.VectorSubcoreMesh(core_axis_name="c", subcore_axis_name="s"),
        out_type=jax.ShapeDtypeStruct((B, D_H), jnp.float32),
        scratch_types=[
            pltpu.VMEM((N_BIG, _BPW), jnp.int32),
            pltpu.VMEM((_SETS, N_BIG, _CHUNK, D_H), jnp.float32),
            pltpu.SemaphoreType.DMA,
            pltpu.SemaphoreType.DMA,
            pltpu.SemaphoreType.DMA,
            pltpu.SemaphoreType.DMA,
        ],
    )
    def _sc_gather_sum(t2_hbm, t3_hbm, xctf_hbm, out_hbm,
                       idx_v, rows_v, sg0, sg1, sg2, so):
        wid = lax.axis_index("s") * 2 + lax.axis_index("c")
        pltpu.sync_copy(xctf_hbm.at[pl.ds(2 * B + wid * _BPW, _BPW)],
                        idx_v.at[0])
        pltpu.sync_copy(xctf_hbm.at[pl.ds(3 * B + wid * _BPW, _BPW)],
                        idx_v.at[1])
        tbls = (t2_hbm, t3_hbm)
        gsems = (sg0, sg1, sg2)
        handles = [None] * _SETS
        out_h = [None] * _SETS

        def issue(ch):
            s = ch % _SETS
            if out_h[s] is not None:
                out_h[s].wait()
                out_h[s] = None
            handles[s] = [
                pltpu.async_copy(
                    tbls[t].at[idx_v.at[t, pl.ds(ch * _CHUNK, _CHUNK)]],
                    rows_v.at[s, t], gsems[s])
                for t in range(N_BIG)
            ]

        for ch in range(min(_SETS, _NCH)):
            issue(ch)
        for ch in range(_NCH):
            s = ch % _SETS
            for h in handles[s]:
                h.wait()

            def _acc_row(r, carry):
                for j in range(D_H // 16):
                    sl = pl.ds(j * 16, 16)
                    rows_v[s, 0, r, sl] = rows_v[s, 0, r, sl] + rows_v[s, 1, r, sl]
                return carry

            lax.fori_loop(0, _CHUNK, _acc_row, 0)
            out_h[s] = pltpu.async_copy(
                rows_v.at[s, 0],
                out_hbm.at[pl.ds(wid * _BPW + ch * _CHUNK, _CHUNK)], so)
            if ch + _SETS < _NCH:
                issue(ch + _SETS)
        for s in range(_SETS):
            if out_h[s] is not None:
                out_h[s].wait()

    return _sc_gather_sum


_MLP_BLK = 8192


_DN_OUT = (((0,), (1,)), ((), ()))


def _mlp_body(g_ref, xct_ref, xnt_ref, waux_ref, b1_ref, w2_ref,
              b2_ref, wo_ref, bo_ref, out_ref):
    xct = xct_ref[...]
    oh0 = (xct[0:1, :] == lax.broadcasted_iota(jnp.int32, (5, _MLP_BLK), 0))
    oh1 = (xct[1:2, :] == lax.broadcasted_iota(jnp.int32, (4, _MLP_BLK), 0))
    aux_t = jnp.concatenate([oh0.astype(jnp.float32), oh1.astype(jnp.float32),
                             xnt_ref[...]], axis=0)
    h = g_ref[...] + lax.dot_general(
        aux_t, waux_ref[...], _DN_01,
        preferred_element_type=jnp.float32) + b1_ref[...]
    h = jnp.maximum(h, 0.0)
    h2 = jnp.dot(h, w2_ref[...], preferred_element_type=jnp.float32) + b2_ref[...]
    h2 = jnp.maximum(h2, 0.0)
    out_ref[...] = lax.dot_general(wo_ref[...], h2, _DN_OUT,
                                   preferred_element_type=jnp.float32) + bo_ref[...]


def _mlp(g, xc_t, xn_t, waux, b1, w2, b2, wo, bo):
    grid = (B // _MLP_BLK,)
    return pl.pallas_call(
        _mlp_body,
        grid=grid,
        in_specs=[
            pl.BlockSpec((_MLP_BLK, D_H), lambda i: (i, 0)),
            pl.BlockSpec((4, _MLP_BLK), lambda i: (0, i)),
            pl.BlockSpec((3, _MLP_BLK), lambda i: (0, i)),
            pl.BlockSpec((D_H, 12), lambda i: (0, 0)),
            pl.BlockSpec((1, D_H), lambda i: (0, 0)),
            pl.BlockSpec((D_H, 64), lambda i: (0, 0)),
            pl.BlockSpec((1, 64), lambda i: (0, 0)),
            pl.BlockSpec((64, 1), lambda i: (0, 0)),
            pl.BlockSpec((1, 1), lambda i: (0, 0)),
        ],
        out_specs=pl.BlockSpec((1, _MLP_BLK), lambda i: (0, i)),
        out_shape=jax.ShapeDtypeStruct((1, B), jnp.float32),
    )(g, xc_t, xn_t, waux, b1, w2, b2, wo, bo)


def kernel(x_cat, x_num, market_emb, ship_emb, order_city_emb,
           customer_city_emb, W1, b1, W2, b2, Wo, bo):
    a_t = order_city_emb.T
    w1t = W1.T
    w1c = W1[9:1808]

    w_aux, p3, extra = _fold_small(market_emb, ship_emb,
                                   customer_city_emb.T, a_t[1792:1799], w1t)
    p2 = _fold_big(a_t, w1c, extra)

    xct = x_cat.astype(jnp.int32).T

    g = _make_sc_gather_sum()(p2, p3, xct.reshape(-1))

    out_t = _mlp(g, xct, x_num.T, w_aux, b1.reshape(1, D_H), W2,
                 b2.reshape(1, 64), Wo, bo.reshape(1, 1))
    return out_t.T
